# baseline (speedup 1.0000x reference)
import jax
import jax.numpy as jnp
from jax.experimental import pallas as pl
from jax.experimental.pallas import tpu as pltpu

THRESH = 0.5
ALPHA = 0.1

def _body(x_ref, y_ref, o_ref):
    @pl.when(pl.program_id(0) == 0)
    def _():
        o_ref[...] = jnp.zeros_like(o_ref)

    xv = x_ref[0]
    yv = y_ref[0]
    N, C = xv.shape
    m = yv[:, 0:1] > THRESH
    lane0 = jax.lax.broadcasted_iota(jnp.int32, (N, C), 1) == 0
    u = jnp.where(m, yv - xv, 0.0)
    v = jnp.where(m, xv, 0.0)
    w = jnp.where(jnp.logical_and(lane0, jnp.logical_not(m)), xv, 0.0)
    s = jnp.sum(u * u, axis=0) - ALPHA * jnp.sum(v * v, axis=0) \
        + ALPHA * jnp.sum(w * w, axis=0)
    o_ref[0, :] += s


def kernel(x, y):
    B, N, C = x.shape
    out = pl.pallas_call(
        _body,
        grid=(B,),
        in_specs=[
            pl.BlockSpec((1, N, C), lambda i: (i, 0, 0)),
            pl.BlockSpec((1, N, C), lambda i: (i, 0, 0)),
        ],
        out_specs=pl.BlockSpec((1, C), lambda i: (0, 0)),
        out_shape=jax.ShapeDtypeStruct((1, C), jnp.float32),
    )(x, y)
    return jnp.sum(out)
